# baseline (speedup 1.0000x reference)
import functools

import jax
import jax.numpy as jnp
from jax import lax
from jax.experimental import pallas as pl
from jax.experimental.pallas import tpu as pltpu
from jax.experimental.pallas import tpu_sc as plsc

_B, _M, _W, _R, _K, _IN = 64, 16384, 64, 8, 8, 1024
_L = 16
_NW = 32
_RPW = _B * _R * _K // _NW
_GPW = _B * _R // _NW


def _proj_body(xi_ref, wrk_ref, brk_ref, wrs_ref, brs_ref, keys_ref, str_ref):
    f32 = jnp.float32
    xi = xi_ref[...]
    keys_ref[...] = jnp.tanh(
        jax.lax.dot_general(xi, wrk_ref[...], (((1,), (1,)), ((), ())),
                            preferred_element_type=f32)
        + brk_ref[...]
    )
    x = (jax.lax.dot_general(xi, wrs_ref[...], (((1,), (1,)), ((), ())),
                             preferred_element_type=f32)
         + brs_ref[...])
    str_ref[...] = jnp.maximum(x, 0.0) + jnp.log(1.0 + jnp.exp(-jnp.abs(x)))


def _knn_body(keys_ref, str_ref, mem_ref, al_ref, ar_ref, pos_ref):
    f32 = jnp.float32
    b = pl.program_id(0)

    keys = keys_ref[0]
    strength = str_ref[0]
    mem = mem_ref[0]

    m2 = jnp.sum(mem * mem, axis=1)
    k2 = jnp.sum(keys * keys, axis=1)
    km = jax.lax.dot_general(keys, mem, (((1,), (1,)), ((), ())),
                             preferred_element_type=f32)
    dist = k2[:, None] + m2[None, :] - 2.0 * km

    iota = jax.lax.broadcasted_iota(jnp.int32, (_R, _M), 1)
    cur = dist
    d_cols, idx_cols = [], []
    for _ in range(_K):
        mv = jnp.min(cur, axis=1, keepdims=True)
        idx = jnp.min(jnp.where(cur == mv, iota, _M),
                      axis=1, keepdims=True)
        d_cols.append(mv)
        idx_cols.append(idx)
        cur = jnp.where(iota == idx, jnp.float32(jnp.inf), cur)

    d = jnp.concatenate(d_cols, axis=1)
    maxd = d[:, _K - 1:_K] + 1e-6
    logits = -(d / maxd) * strength
    logits = logits - jnp.max(logits, axis=1, keepdims=True)
    e = jnp.exp(logits)
    attn = e / jnp.sum(e, axis=1, keepdims=True)

    pos = jnp.concatenate(idx_cols, axis=1) + b * _M
    par = (pos & 1).astype(f32)
    al_ref[0] = attn * (1.0 - par)
    ar_ref[0] = attn * par
    pos_ref[0] = lax.shift_right_logical(pos, 1)


def _sc_combine(table_hbm, idx_hbm, al_hbm, ar_hbm, out_hbm,
                idx_v, rows_v, al_v, ar_v, out_v, sem):
    wid = lax.axis_index("s") * 2 + lax.axis_index("c")
    rbase = wid * _RPW
    pltpu.sync_copy(idx_hbm.at[pl.ds(rbase, _RPW)], idx_v)
    pltpu.sync_copy(al_hbm.at[pl.ds(rbase, _RPW)], al_v)
    pltpu.sync_copy(ar_hbm.at[pl.ds(rbase, _RPW)], ar_v)
    pltpu.async_copy(table_hbm.at[idx_v], rows_v, sem).wait()
    for g in range(_GPW):
        als = [al_v[g * _K + k, :] for k in range(_K)]
        ars = [ar_v[g * _K + k, :] for k in range(_K)]
        for c in range(_W // _L):
            j = g * _K
            acc = (als[0] * rows_v[j, pl.ds(c * _L, _L)]
                   + ars[0] * rows_v[j, pl.ds(_W + c * _L, _L)])
            for k in range(1, _K):
                acc = (acc + als[k] * rows_v[j + k, pl.ds(c * _L, _L)]
                       + ars[k] * rows_v[j + k, pl.ds(_W + c * _L, _L)])
            out_v[g, pl.ds(c * _L, _L)] = acc
    pltpu.sync_copy(out_v, out_hbm.at[pl.ds(wid * _GPW, _GPW)])


@jax.jit
def kernel(xi, memory, W_rk, b_rk, W_rs, b_rs):
    f32 = jnp.float32
    keys_flat, strengths = pl.pallas_call(
        _proj_body,
        out_shape=[
            jax.ShapeDtypeStruct((_B, _R * _W), f32),
            jax.ShapeDtypeStruct((_B, _R), f32),
        ],
    )(xi, W_rk, b_rk.reshape(1, _R * _W), W_rs, b_rs.reshape(1, _R))

    keys = keys_flat.reshape(_B, _R, _W)
    strengths3 = strengths.reshape(_B, _R, 1)

    attn_l, attn_r, pos = pl.pallas_call(
        _knn_body,
        grid=(_B,),
        in_specs=[
            pl.BlockSpec((1, _R, _W), lambda b: (b, 0, 0)),
            pl.BlockSpec((1, _R, 1), lambda b: (b, 0, 0)),
            pl.BlockSpec((1, _M, _W), lambda b: (b, 0, 0)),
        ],
        out_specs=[
            pl.BlockSpec((1, _R, _K), lambda b: (b, 0, 0)),
            pl.BlockSpec((1, _R, _K), lambda b: (b, 0, 0)),
            pl.BlockSpec((1, _R, _K), lambda b: (b, 0, 0)),
        ],
        out_shape=[
            jax.ShapeDtypeStruct((_B, _R, _K), f32),
            jax.ShapeDtypeStruct((_B, _R, _K), f32),
            jax.ShapeDtypeStruct((_B, _R, _K), jnp.int32),
        ],
        compiler_params=pltpu.CompilerParams(
            dimension_semantics=("arbitrary",),
        ),
    )(keys, strengths3, memory)

    table = memory.reshape(_B * _M // 2, 2 * _W)
    n = _B * _R * _K
    idx_flat = pos.reshape(n)
    al_bc = jnp.broadcast_to(attn_l.reshape(n, 1), (n, _L))
    ar_bc = jnp.broadcast_to(attn_r.reshape(n, 1), (n, _L))

    mesh = plsc.VectorSubcoreMesh(core_axis_name="c", subcore_axis_name="s")
    sc = functools.partial(
        pl.kernel,
        mesh=mesh,
        out_type=jax.ShapeDtypeStruct((_B * _R, _W), f32),
        scratch_types=[
            pltpu.VMEM((_RPW,), jnp.int32),
            pltpu.VMEM((_RPW, 2 * _W), f32),
            pltpu.VMEM((_RPW, _L), f32),
            pltpu.VMEM((_RPW, _L), f32),
            pltpu.VMEM((_GPW, _W), f32),
            pltpu.SemaphoreType.DMA,
        ],
    )(_sc_combine)
    read = sc(table, idx_flat, al_bc, ar_bc)
    return read.reshape(_B, _R, _W)


# speedup vs baseline: 1.3343x; 1.3343x over previous
import jax
import jax.numpy as jnp
from jax import lax
from jax.experimental import pallas as pl
from jax.experimental.pallas import tpu as pltpu

_B, _M, _W, _R, _K, _IN = 64, 16384, 64, 8, 8, 1024


def _proj_body(xi_ref, wrk_ref, brk_ref, wrs_ref, brs_ref, keys_ref, str_ref):
    f32 = jnp.float32
    xi = xi_ref[...]
    keys_ref[...] = jnp.tanh(
        jax.lax.dot_general(xi, wrk_ref[...], (((1,), (1,)), ((), ())),
                            preferred_element_type=f32)
        + brk_ref[...]
    )
    x = (jax.lax.dot_general(xi, wrs_ref[...], (((1,), (1,)), ((), ())),
                             preferred_element_type=f32)
         + brs_ref[...])
    str_ref[...] = jnp.maximum(x, 0.0) + jnp.log(1.0 + jnp.exp(-jnp.abs(x)))


def _knn_body(keys_ref, str_ref, mem_hbm, out_ref, buf, sem):
    f32 = jnp.float32
    b = pl.program_id(0)

    slot = jax.lax.rem(b, 2)

    @pl.when(b == 0)
    def _():
        pltpu.make_async_copy(mem_hbm.at[0], buf.at[0], sem.at[0]).start()

    @pl.when(b + 1 < _B)
    def _():
        nxt = jax.lax.rem(b + 1, 2)
        pltpu.make_async_copy(mem_hbm.at[b + 1], buf.at[nxt],
                              sem.at[nxt]).start()

    pltpu.make_async_copy(mem_hbm.at[b], buf.at[slot], sem.at[slot]).wait()

    keys = keys_ref[0]
    strength = str_ref[0]
    mem = buf[slot]

    m2 = jnp.sum(mem * mem, axis=1)
    k2 = jnp.sum(keys * keys, axis=1)
    km = jax.lax.dot_general(keys, mem, (((1,), (1,)), ((), ())),
                             preferred_element_type=f32)
    dist = k2[:, None] + m2[None, :] - 2.0 * km

    iota = jax.lax.broadcasted_iota(jnp.int32, (_R, _M), 1)
    cur = dist
    d_cols, idx_cols = [], []
    for _ in range(_K):
        mv = jnp.min(cur, axis=1, keepdims=True)
        idx = jnp.min(jnp.where(cur == mv, iota, _M),
                      axis=1, keepdims=True)
        d_cols.append(mv)
        idx_cols.append(idx)
        cur = jnp.where(iota == idx, jnp.float32(jnp.inf), cur)

    d = jnp.concatenate(d_cols, axis=1)
    maxd = d[:, _K - 1:_K] + 1e-6
    logits = -(d / maxd) * strength
    logits = logits - jnp.max(logits, axis=1, keepdims=True)
    e = jnp.exp(logits)
    attn = e / jnp.sum(e, axis=1, keepdims=True)

    wmat = jnp.zeros((_R, _M), f32)
    for k in range(_K):
        wmat = wmat + jnp.where(iota == idx_cols[k], attn[:, k:k + 1], 0.0)

    bf16 = jnp.bfloat16
    w_hi = wmat.astype(bf16).astype(f32)
    w_lo = wmat - w_hi
    m_hi = mem.astype(bf16).astype(f32)
    m_lo = mem - m_hi
    dn = (((1,), (0,)), ((), ()))
    read = (jax.lax.dot_general(w_hi, m_hi, dn, preferred_element_type=f32)
            + jax.lax.dot_general(w_hi, m_lo, dn, preferred_element_type=f32)
            + jax.lax.dot_general(w_lo, m_hi, dn, preferred_element_type=f32))
    out_ref[0] = read


@jax.jit
def kernel(xi, memory, W_rk, b_rk, W_rs, b_rs):
    f32 = jnp.float32
    keys_flat, strengths = pl.pallas_call(
        _proj_body,
        out_shape=[
            jax.ShapeDtypeStruct((_B, _R * _W), f32),
            jax.ShapeDtypeStruct((_B, _R), f32),
        ],
    )(xi, W_rk, b_rk.reshape(1, _R * _W), W_rs, b_rs.reshape(1, _R))

    keys = keys_flat.reshape(_B, _R, _W)
    strengths3 = strengths.reshape(_B, _R, 1)

    out = pl.pallas_call(
        _knn_body,
        grid=(_B,),
        in_specs=[
            pl.BlockSpec((1, _R, _W), lambda b: (b, 0, 0)),
            pl.BlockSpec((1, _R, 1), lambda b: (b, 0, 0)),
            pl.BlockSpec(memory_space=pl.ANY),
        ],
        scratch_shapes=[
            pltpu.VMEM((2, _M, _W), jnp.float32),
            pltpu.SemaphoreType.DMA((2,)),
        ],
        out_specs=pl.BlockSpec((1, _R, _W), lambda b: (b, 0, 0)),
        out_shape=jax.ShapeDtypeStruct((_B, _R, _W), f32),
        compiler_params=pltpu.CompilerParams(
            dimension_semantics=("arbitrary",),
        ),
    )(keys, strengths3, memory)
    return out


# speedup vs baseline: 1.5150x; 1.1355x over previous
import functools

import jax
import jax.numpy as jnp
from jax import lax
from jax.experimental import pallas as pl
from jax.experimental.pallas import tpu as pltpu
from jax.experimental.pallas import tpu_sc as plsc

_B, _M, _W, _R, _K, _IN = 64, 16384, 64, 8, 8, 1024
_L = 16
_NW = 32
_RPW = _B * _R * _K // _NW
_GPW = _B * _R // _NW


def _proj_body(xi_ref, wrk_ref, brk_ref, wrs_ref, brs_ref, keys_ref, str_ref):
    f32 = jnp.float32
    xi = xi_ref[...]
    keys_ref[...] = jnp.tanh(
        jax.lax.dot_general(xi, wrk_ref[...], (((1,), (1,)), ((), ())),
                            preferred_element_type=f32)
        + brk_ref[...]
    )
    x = (jax.lax.dot_general(xi, wrs_ref[...], (((1,), (1,)), ((), ())),
                             preferred_element_type=f32)
         + brs_ref[...])
    str_ref[...] = jnp.maximum(x, 0.0) + jnp.log(1.0 + jnp.exp(-jnp.abs(x)))


def _knn_body(keys_ref, str_ref, mem_hbm, al_ref, ar_ref, pos_ref,
              tab_ref, buf, sem):
    f32 = jnp.float32
    b = pl.program_id(0)

    slot = jax.lax.rem(b, 2)
    nq = 4
    qs = _M // nq

    def _start(bb, sl):
        for q in range(nq):
            pltpu.make_async_copy(
                mem_hbm.at[bb, pl.ds(q * qs, qs)],
                buf.at[sl, pl.ds(q * qs, qs)],
                sem.at[sl, q]).start()

    @pl.when(b == 0)
    def _():
        _start(0, 0)

    @pl.when(b + 1 < _B)
    def _():
        _start(b + 1, jax.lax.rem(b + 1, 2))

    for q in range(nq):
        pltpu.make_async_copy(
            mem_hbm.at[b, pl.ds(q * qs, qs)],
            buf.at[slot, pl.ds(q * qs, qs)],
            sem.at[slot, q]).wait()

    keys = keys_ref[0]
    strength = str_ref[0]
    mem = buf[slot]

    m2 = jnp.sum(mem * mem, axis=1)
    k2 = jnp.sum(keys * keys, axis=1)
    km = jax.lax.dot_general(keys, mem, (((1,), (1,)), ((), ())),
                             preferred_element_type=f32)
    dist = k2[:, None] + m2[None, :] - 2.0 * km

    iota = jax.lax.broadcasted_iota(jnp.int32, (_R, _M), 1)
    cur = dist
    d_cols, idx_cols = [], []
    for _ in range(_K):
        mv = jnp.min(cur, axis=1, keepdims=True)
        idx = jnp.min(jnp.where(cur == mv, iota, _M),
                      axis=1, keepdims=True)
        d_cols.append(mv)
        idx_cols.append(idx)
        cur = jnp.where(iota == idx, jnp.float32(jnp.inf), cur)

    d = jnp.concatenate(d_cols, axis=1)
    maxd = d[:, _K - 1:_K] + 1e-6
    logits = -(d / maxd) * strength
    logits = logits - jnp.max(logits, axis=1, keepdims=True)
    e = jnp.exp(logits)
    attn = e / jnp.sum(e, axis=1, keepdims=True)

    h = _M // 2
    tab_ref[0] = jnp.concatenate([mem[:h, :], mem[h:, :]], axis=1)

    idx = jnp.concatenate(idx_cols, axis=1)
    par = lax.shift_right_logical(idx, 13).astype(f32)
    al_ref[0] = attn * (1.0 - par)
    ar_ref[0] = attn * par
    pos_ref[0] = (idx & (h - 1)) + b * h


def _sc_combine(table_hbm, idx_hbm, al_hbm, ar_hbm, out_hbm,
                idx_v, rows_v, al_v, ar_v, out_v, sem):
    wid = lax.axis_index("s") * 2 + lax.axis_index("c")
    rbase = wid * _RPW
    pltpu.sync_copy(idx_hbm.at[pl.ds(rbase, _RPW)], idx_v)
    pltpu.sync_copy(al_hbm.at[pl.ds(rbase, _RPW)], al_v)
    pltpu.sync_copy(ar_hbm.at[pl.ds(rbase, _RPW)], ar_v)
    pltpu.async_copy(table_hbm.at[idx_v], rows_v, sem).wait()
    for g in range(_GPW):
        als = [al_v[g * _K + k, :] for k in range(_K)]
        ars = [ar_v[g * _K + k, :] for k in range(_K)]
        for c in range(_W // _L):
            j = g * _K
            acc = (als[0] * rows_v[j, pl.ds(c * _L, _L)]
                   + ars[0] * rows_v[j, pl.ds(_W + c * _L, _L)])
            for k in range(1, _K):
                acc = (acc + als[k] * rows_v[j + k, pl.ds(c * _L, _L)]
                       + ars[k] * rows_v[j + k, pl.ds(_W + c * _L, _L)])
            out_v[g, pl.ds(c * _L, _L)] = acc
    pltpu.sync_copy(out_v, out_hbm.at[pl.ds(wid * _GPW, _GPW)])


@jax.jit
def kernel(xi, memory, W_rk, b_rk, W_rs, b_rs):
    f32 = jnp.float32
    keys_flat, strengths = pl.pallas_call(
        _proj_body,
        out_shape=[
            jax.ShapeDtypeStruct((_B, _R * _W), f32),
            jax.ShapeDtypeStruct((_B, _R), f32),
        ],
    )(xi, W_rk, b_rk.reshape(1, _R * _W), W_rs, b_rs.reshape(1, _R))

    keys = keys_flat.reshape(_B, _R, _W)
    strengths3 = strengths.reshape(_B, _R, 1)

    attn_l, attn_r, pos, tab = pl.pallas_call(
        _knn_body,
        grid=(_B,),
        in_specs=[
            pl.BlockSpec((1, _R, _W), lambda b: (b, 0, 0)),
            pl.BlockSpec((1, _R, 1), lambda b: (b, 0, 0)),
            pl.BlockSpec(memory_space=pl.ANY),
        ],
        scratch_shapes=[
            pltpu.VMEM((2, _M, _W), jnp.float32),
            pltpu.SemaphoreType.DMA((2, 4)),
        ],
        out_specs=[
            pl.BlockSpec((1, _R, _K), lambda b: (b, 0, 0)),
            pl.BlockSpec((1, _R, _K), lambda b: (b, 0, 0)),
            pl.BlockSpec((1, _R, _K), lambda b: (b, 0, 0)),
            pl.BlockSpec((1, _M // 2, 2 * _W), lambda b: (b, 0, 0)),
        ],
        out_shape=[
            jax.ShapeDtypeStruct((_B, _R, _K), f32),
            jax.ShapeDtypeStruct((_B, _R, _K), f32),
            jax.ShapeDtypeStruct((_B, _R, _K), jnp.int32),
            jax.ShapeDtypeStruct((_B, _M // 2, 2 * _W), f32),
        ],
        compiler_params=pltpu.CompilerParams(
            dimension_semantics=("arbitrary",),
        ),
    )(keys, strengths3, memory)

    table = tab.reshape(_B * _M // 2, 2 * _W)
    n = _B * _R * _K
    idx_flat = pos.reshape(n)
    al_bc = jnp.broadcast_to(attn_l.reshape(n, 1), (n, _L))
    ar_bc = jnp.broadcast_to(attn_r.reshape(n, 1), (n, _L))

    mesh = plsc.VectorSubcoreMesh(core_axis_name="c", subcore_axis_name="s")
    sc = functools.partial(
        pl.kernel,
        mesh=mesh,
        out_type=jax.ShapeDtypeStruct((_B * _R, _W), f32),
        scratch_types=[
            pltpu.VMEM((_RPW,), jnp.int32),
            pltpu.VMEM((_RPW, 2 * _W), f32),
            pltpu.VMEM((_RPW, _L), f32),
            pltpu.VMEM((_RPW, _L), f32),
            pltpu.VMEM((_GPW, _W), f32),
            pltpu.SemaphoreType.DMA,
        ],
    )(_sc_combine)
    read = sc(table, idx_flat, al_bc, ar_bc)
    return read.reshape(_B, _R, _W)
